# baseline (speedup 1.0000x reference)
import jax
import jax.numpy as jnp
from jax import lax
from jax.experimental import pallas as pl
from jax.experimental.pallas import tpu as pltpu
from jax.experimental.pallas import tpu_sc as plsc

_NUM_EMBEDDINGS = 1000000
_EMBEDDING_DIM = 32
_BATCH = 16384
_FIELDS = 26

_INFO = plsc.get_sparse_core_info()
_NC = _INFO.num_cores
_NS = _INFO.num_subcores
_NW = _NC * _NS
_BW = _BATCH // _NW
_NBUF = 2


def _body(idx_hbm, table_hbm, out_hbm, *scratch):
    idx_v = scratch[0:_NBUF]
    rows_v = scratch[_NBUF:2 * _NBUF]
    rowsT_v = scratch[2 * _NBUF:3 * _NBUF]
    gsem = scratch[3 * _NBUF:4 * _NBUF]
    wsem = scratch[4 * _NBUF:5 * _NBUF]

    wid = lax.axis_index("s") * _NC + lax.axis_index("c")
    b0 = wid * _BW

    def start_gather(f, p):
        pltpu.sync_copy(idx_hbm.at[pl.ds(f * _BATCH + b0, _BW)], idx_v[p])
        pltpu.async_copy(table_hbm.at[idx_v[p]], rows_v[p], gsem[p])

    def transpose(p):
        lanes = lax.iota(jnp.int32, 16)

        def per_d(d, _):
            col = jnp.full((16,), 0, jnp.int32) + d
            for bb in range(_BW // 16):
                row = lanes + (bb * 16)
                v = plsc.load_gather(rows_v[p], [row, col])
                rowsT_v[p][d, pl.ds(bb * 16, 16)] = v
            return ()

        lax.fori_loop(0, _EMBEDDING_DIM, per_d, ())

    def start_write(f, p):
        pltpu.async_copy(rowsT_v[p], out_hbm.at[f, :, pl.ds(b0, _BW)], wsem[p])

    def wait_gather(p):
        pltpu.make_async_copy(table_hbm.at[idx_v[p]], rows_v[p], gsem[p]).wait()

    def wait_write(f, p):
        pltpu.make_async_copy(rowsT_v[p], out_hbm.at[f, :, pl.ds(b0, _BW)], wsem[p]).wait()

    start_gather(0, 0)
    start_gather(1, 1)

    def group(g, _):
        for p in range(_NBUF):
            f = g * _NBUF + p
            wait_gather(p)
            @pl.when(f >= _NBUF)
            def _():
                wait_write(f - _NBUF, p)
            transpose(p)
            start_write(f, p)
            nxt = f + _NBUF
            @pl.when(nxt < _FIELDS)
            def _():
                start_gather(nxt, p)
        return ()

    lax.fori_loop(0, _FIELDS // _NBUF, group, ())
    for p in range(_NBUF):
        wait_write(_FIELDS - _NBUF + p, p)


@jax.jit
def kernel(input_, weight):
    idx_flat = jnp.transpose(input_).reshape(_FIELDS * _BATCH)
    mesh = plsc.VectorSubcoreMesh(core_axis_name="c", subcore_axis_name="s")
    out_t = pl.kernel(
        _body,
        out_type=jax.ShapeDtypeStruct((_FIELDS, _EMBEDDING_DIM, _BATCH), jnp.float32),
        mesh=mesh,
        scratch_types=(
            [pltpu.VMEM((_BW,), jnp.int32) for _ in range(_NBUF)]
            + [pltpu.VMEM((_BW, _EMBEDDING_DIM), jnp.float32) for _ in range(_NBUF)]
            + [pltpu.VMEM((_EMBEDDING_DIM, _BW), jnp.float32) for _ in range(_NBUF)]
            + [pltpu.SemaphoreType.DMA for _ in range(2 * _NBUF)]
        ),
        compiler_params=pltpu.CompilerParams(
            use_tc_tiling_on_sc=False, needs_layout_passes=False
        ),
    )(idx_flat, weight)
    return jnp.transpose(out_t, (2, 0, 1))


# speedup vs baseline: 1.3997x; 1.3997x over previous
import jax
import jax.numpy as jnp
from jax import lax
from jax.experimental import pallas as pl
from jax.experimental.pallas import tpu as pltpu
from jax.experimental.pallas import tpu_sc as plsc

_NUM_EMBEDDINGS = 1000000
_EMBEDDING_DIM = 32
_BATCH = 16384
_FIELDS = 26

_INFO = plsc.get_sparse_core_info()
_NC = _INFO.num_cores
_NS = _INFO.num_subcores
_NW = _NC * _NS
_BW = _BATCH // _NW
_NBUF = 2
_PITCH = _BW + 1
_UNROLL = 8


def _body(idx_hbm, table_hbm, out_hbm, *scratch):
    idx_v = scratch[0:_NBUF]
    rows_v = scratch[_NBUF:2 * _NBUF]
    rowsT_v = scratch[2 * _NBUF:3 * _NBUF]
    gsem = scratch[3 * _NBUF:4 * _NBUF]
    wsem = scratch[4 * _NBUF:5 * _NBUF]

    wid = lax.axis_index("s") * _NC + lax.axis_index("c")
    b0 = wid * _BW

    def start_gather(f, p):
        pltpu.sync_copy(idx_hbm.at[pl.ds(f * _BATCH + b0, _BW)], idx_v[p])
        pltpu.async_copy(table_hbm.at[idx_v[p]], rows_v[p], gsem[p])

    def transpose(p):
        lanes = lax.iota(jnp.int32, 16)
        lanes_hi = lanes + 16
        zero = jnp.zeros((16,), jnp.int32)
        tgt = rowsT_v[p]

        def per_rows(jj, _):
            for u in range(_UNROLL):
                j = jj * _UNROLL + u
                col = zero + j
                v0 = rows_v[p][j, pl.ds(0, 16)]
                v1 = rows_v[p][j, pl.ds(16, 16)]
                plsc.store_scatter(tgt, [lanes, col], v0)
                plsc.store_scatter(tgt, [lanes_hi, col], v1)
            return ()

        lax.fori_loop(0, _BW // _UNROLL, per_rows, ())

    def start_write(f, p):
        pltpu.async_copy(
            rowsT_v[p].at[:, pl.ds(0, _BW)], out_hbm.at[f, :, pl.ds(b0, _BW)], wsem[p]
        )

    def wait_gather(p):
        pltpu.make_async_copy(table_hbm.at[idx_v[p]], rows_v[p], gsem[p]).wait()

    def wait_write(f, p):
        pltpu.make_async_copy(
            rowsT_v[p].at[:, pl.ds(0, _BW)], out_hbm.at[f, :, pl.ds(b0, _BW)], wsem[p]
        ).wait()

    start_gather(0, 0)
    start_gather(1, 1)

    def group(g, _):
        for p in range(_NBUF):
            f = g * _NBUF + p
            wait_gather(p)
            @pl.when(f >= _NBUF)
            def _():
                wait_write(f - _NBUF, p)
            transpose(p)
            start_write(f, p)
            nxt = f + _NBUF
            @pl.when(nxt < _FIELDS)
            def _():
                start_gather(nxt, p)
        return ()

    lax.fori_loop(0, _FIELDS // _NBUF, group, ())
    for p in range(_NBUF):
        wait_write(_FIELDS - _NBUF + p, p)


@jax.jit
def kernel(input_, weight):
    idx_flat = jnp.transpose(input_).reshape(_FIELDS * _BATCH)
    mesh = plsc.VectorSubcoreMesh(core_axis_name="c", subcore_axis_name="s")
    out_t = pl.kernel(
        _body,
        out_type=jax.ShapeDtypeStruct((_FIELDS, _EMBEDDING_DIM, _BATCH), jnp.float32),
        mesh=mesh,
        scratch_types=(
            [pltpu.VMEM((_BW,), jnp.int32) for _ in range(_NBUF)]
            + [pltpu.VMEM((_BW, _EMBEDDING_DIM), jnp.float32) for _ in range(_NBUF)]
            + [pltpu.VMEM((_EMBEDDING_DIM, _PITCH), jnp.float32) for _ in range(_NBUF)]
            + [pltpu.SemaphoreType.DMA for _ in range(2 * _NBUF)]
        ),
        compiler_params=pltpu.CompilerParams(
            use_tc_tiling_on_sc=False, needs_layout_passes=False
        ),
    )(idx_flat, weight)
    return jnp.transpose(out_t, (2, 0, 1))
